# baseline (speedup 1.0000x reference)
import numpy as np

import jax
import jax.numpy as jnp
from jax import lax
from jax.experimental import pallas as pl
from jax.experimental.pallas import tpu as pltpu
from jax.experimental.pallas import tpu_sc as plsc

B = 4
N = 8192
S = 1024
NSAMPLE = 32
C = 128
NCH = C + 3
RAD2 = 0.2 * 0.2

NC = 2
NSUB = 16
L = 16
NW = NC * NSUB
WPB = NW // B
QPW = S // WPB
CPW = (NCH + WPB - 1) // WPB

NWRD = N // 16
PCHK = 2048
NCHK = N // PCHK
QT = 128
NQT = S // QT

_WPACK = np.where(
    (np.arange(PCHK)[:, None] // 16) == np.arange(PCHK // 16)[None, :],
    np.exp2(np.arange(PCHK) % 16)[:, None],
    0.0,
).astype(np.float32)


def _mesh():
    return plsc.VectorSubcoreMesh(
        core_axis_name="c", subcore_axis_name="s", num_cores=NC, num_subcores=NSUB
    )


def _mask_body(px, py, pz, qxr, qyr, qzr, wmat, out, msk):
    qv = [(qxr[0, qs], qyr[0, qs], qzr[0, qs]) for qs in range(QT // 8)]
    for c in range(NCHK):
        for t in range(PCHK // 128):
            o = c * PCHK + t * 128
            pxv = px[0, 0, pl.ds(o, 128)][None, :]
            pyv = py[0, 0, pl.ds(o, 128)][None, :]
            pzv = pz[0, 0, pl.ds(o, 128)][None, :]
            for qs in range(QT // 8):
                qxv, qyv, qzv = qv[qs]
                dx = qxv - pxv
                dy = qyv - pyv
                dz = qzv - pzv
                d = dx * dx + dy * dy + dz * dz
                msk[qs * 8:(qs + 1) * 8, t * 128:(t + 1) * 128] = jnp.where(
                    d < RAD2, 1.0, 0.0)
        words = jnp.dot(msk[...], wmat[...],
                        preferred_element_type=jnp.float32)
        out[0, :, pl.ds(c * (PCHK // 16), PCHK // 16)] = words.astype(jnp.int32)


def _mask(px, py, pz, qxr, qyr, qzr, wmat):
    pspec = pl.BlockSpec((1, 1, N), lambda b, qt: (b, 0, 0))
    qspec = pl.BlockSpec((1, QT // 8, 8, 1), lambda b, qt: (b, qt, 0, 0))
    return pl.pallas_call(
        _mask_body,
        grid=(B, NQT),
        in_specs=[pspec, pspec, pspec, qspec, qspec, qspec,
                  pl.BlockSpec((PCHK, PCHK // 16), lambda b, qt: (0, 0))],
        out_specs=pl.BlockSpec((1, QT, NWRD), lambda b, qt: (b, qt, 0)),
        out_shape=jax.ShapeDtypeStruct((B, S, NWRD), jnp.int32),
        scratch_shapes=[pltpu.VMEM((QT, PCHK), jnp.float32)],
    )(px, py, pz, qxr, qyr, qzr, wmat)


def _sg_body(words, xt3, ft, nxt, out, wbuf, cand, obuf, ib, rb, ob, cb, shr):
    w = lax.axis_index("c") * NSUB + lax.axis_index("s")
    b = w // WPB
    bl = b % 2
    qc = w % WPB
    g = w % WPB
    lanes = lax.iota(jnp.int32, L)

    HQ = QPW // 2
    for h2 in range(2):
        pltpu.sync_copy(words.at[b, pl.ds(qc * QPW + h2 * HQ, HQ)], wbuf)

        def per_q(qi, carry):
            def grp(gg, cnt):
                off = pl.multiple_of(gg * L, L)
                wv = wbuf[qi, pl.ds(off, L)]
                mv0 = jnp.where(jnp.logical_and(wv != 0, cnt < NSAMPLE), 1, 0)

                def cond(state):
                    mv, cnt2 = state
                    any_nz = plsc.all_reduce_population_count(mv == 1)[0]
                    return jnp.logical_and(any_nz > 0, cnt2 < NSAMPLE)

                def expand(state):
                    mv, cnt2 = state
                    l = plsc.all_reduce_ffs(mv == 1)[0]
                    widx = gg * L + l
                    wsp = plsc.load_gather(
                        wbuf, [jnp.full((L,), qi, jnp.int32),
                               jnp.full((L,), widx, jnp.int32)])
                    bits = (wsp >> lanes) & 1
                    ranks = cnt2 + plsc.cumsum(bits) - 1
                    mm = jnp.logical_and(bits == 1, ranks < NSAMPLE)
                    plsc.store_scatter(cand, [ranks], widx * L + lanes, mask=mm)
                    cnt3 = cnt2 + jnp.sum(bits)
                    return jnp.where(lanes == l, 0, mv), cnt3

                return lax.while_loop(cond, expand, (mv0, cnt))[1]

            cnt = lax.fori_loop(0, NWRD // L, grp, jnp.int32(0))
            first = cand[pl.ds(0, L)][0]
            fill = jnp.where(cnt > 0, first, 0)
            for hh in range(NSAMPLE // L):
                slots = hh * L + lanes
                cur = cand[pl.ds(hh * L, L)]
                off = pl.multiple_of((h2 * HQ + qi) * NSAMPLE + hh * L, L)
                obuf[pl.ds(off, L)] = jnp.where(slots < cnt, cur, fill)
            return carry

        lax.fori_loop(0, HQ, per_q, 0)

    pltpu.sync_copy(obuf, shr.at[bl, pl.ds(qc * QPW * NSAMPLE, QPW * NSAMPLE)])
    plsc.subcore_barrier()
    pltpu.sync_copy(shr.at[bl], ib)

    pltpu.sync_copy(nxt.at[b], cb)
    nv = S * NSAMPLE // L

    def per_chan(j, carry):
        ch = g + WPB * j

        @pl.when(ch < 3)
        def _():
            pltpu.sync_copy(xt3.at[b * 3 + ch], rb)

        @pl.when(jnp.logical_and(ch >= 3, ch < NCH))
        def _():
            pltpu.sync_copy(ft.at[b * C + ch - 3], rb)

        @pl.when(ch < NCH)
        def _():
            zv = jnp.zeros((L,), jnp.int32)

            def gstep(v, c2):
                off = pl.multiple_of(v * L, L)
                ivec = ib[pl.ds(off, L)]
                ob[0, pl.ds(off, L)] = plsc.load_gather(rb, [zv, ivec])
                return c2

            lax.fori_loop(0, nv, gstep, 0)

            @pl.when(ch < 3)
            def _():
                crow = jnp.where(ch < 3, ch, 0)

                def sgrp(sg, c3):
                    soff = pl.multiple_of(sg * L, L)
                    ctrv = cb[crow, pl.ds(soff, L)]
                    for i in range(L):
                        ctr = ctrv[i]
                        p = (sg * L + i) * NSAMPLE
                        for hh in range(NSAMPLE // L):
                            off = pl.multiple_of(p + hh * L, L)
                            ob[0, pl.ds(off, L)] = ob[0, pl.ds(off, L)] - ctr
                    return c3

                lax.fori_loop(0, S // L, sgrp, 0)

            pltpu.sync_copy(ob, out.at[b * NCH + ch])

        return carry

    lax.fori_loop(0, CPW, per_chan, 0)


def _selgroup(words, xt3, ft, nxt):
    return pl.kernel(
        _sg_body,
        out_type=jax.ShapeDtypeStruct((B * NCH, 1, S * NSAMPLE), jnp.float32),
        mesh=_mesh(),
        scratch_types=[
            pltpu.VMEM((QPW // 2, NWRD), jnp.int32),
            pltpu.VMEM((NSAMPLE,), jnp.int32),
            pltpu.VMEM((QPW * NSAMPLE,), jnp.int32),
            pltpu.VMEM((S * NSAMPLE,), jnp.int32),
            pltpu.VMEM((1, N), jnp.float32),
            pltpu.VMEM((1, S * NSAMPLE), jnp.float32),
            pltpu.VMEM((3, S), jnp.float32),
            pltpu.VMEM_SHARED((2, S * NSAMPLE), jnp.int32),
        ],
        compiler_params=pltpu.CompilerParams(needs_layout_passes=False),
    )(words, xt3, ft, nxt)


def kernel(xyz, new_xyz, features):
    px = xyz[..., 0].reshape(B, 1, N)
    py = xyz[..., 1].reshape(B, 1, N)
    pz = xyz[..., 2].reshape(B, 1, N)
    qxr = new_xyz[..., 0].reshape(B, S // 8, 8, 1)
    qyr = new_xyz[..., 1].reshape(B, S // 8, 8, 1)
    qzr = new_xyz[..., 2].reshape(B, S // 8, 8, 1)
    wmat = jnp.asarray(_WPACK)
    words = _mask(px, py, pz, qxr, qyr, qzr, wmat)
    xt3 = jnp.transpose(xyz, (0, 2, 1)).reshape(B * 3, 1, N)
    ft = features.reshape(B * C, 1, N)
    nxt = jnp.transpose(new_xyz, (0, 2, 1))
    out = _selgroup(words, xt3, ft, nxt)
    return out.reshape(B, NCH, S, NSAMPLE)
